# Initial kernel scaffold; baseline (speedup 1.0000x reference)
import jax
import jax.numpy as jnp
from jax.experimental import pallas as pl


def kernel(h, group_idx, batch, W1_root, W1_nbr, b1, W3_root, W3_nbr, b3):
    raise NotImplementedError("write your pallas kernel here")


# speedup vs baseline: 5.1530x; 5.1530x over previous
import functools

import jax
import jax.numpy as jnp
from jax import lax
from jax.experimental import pallas as pl
from jax.experimental.pallas import tpu as pltpu
from jax.experimental.pallas import tpu_sc as plsc

G = 8
NC = 2
NS = 16
CB = 128


def _make_segsum(n_pad, D, ch):
  mesh = plsc.VectorSubcoreMesh(
      core_axis_name="c", subcore_axis_name="s", num_cores=NC, num_subcores=NS)
  rows_per_sub = n_pad // NS

  @functools.partial(
      pl.kernel,
      out_type=jax.ShapeDtypeStruct((NC, n_pad, D), jnp.float32),
      mesh=mesh,
      scratch_types=[
          pltpu.VMEM((ch, CB), jnp.int32),
          pltpu.VMEM((ch, CB), jnp.int32),
          pltpu.VMEM((CB, D), jnp.float32),
          pltpu.VMEM_SHARED((n_pad, D), jnp.float32),
          pltpu.SemaphoreType.DMA,
      ],
  )
  def segsum(x_hbm, zeros_hbm, src_hbm, dst_hbm, out_hbm,
             src_v, dst_v, rows_v, acc, sem):
    c = lax.axis_index("c")
    s = lax.axis_index("s")
    w = s * NC + c
    r0 = s * rows_per_sub
    pltpu.sync_copy(zeros_hbm, acc.at[pl.ds(r0, rows_per_sub)])
    pltpu.sync_copy(src_hbm.at[w], src_v)
    pltpu.sync_copy(dst_hbm.at[w], dst_v)
    plsc.subcore_barrier()

    def step(k, carry):
      pltpu.async_copy(x_hbm.at[src_v.at[k]], rows_v, sem).wait()
      pltpu.sync_copy(rows_v, acc.at[dst_v.at[k]], add=True)
      return carry

    lax.fori_loop(0, ch, step, 0)
    plsc.subcore_barrier()
    pltpu.sync_copy(acc.at[pl.ds(r0, rows_per_sub)],
                    out_hbm.at[c, pl.ds(r0, rows_per_sub)])

  return segsum


def _l1_body(p_ref, h_ref, wr_ref, wn_ref, b_ref, o_ref):
  a = p_ref[0] + p_ref[1]
  z = (jnp.dot(h_ref[...], wr_ref[...], preferred_element_type=jnp.float32)
       + jnp.dot(a, wn_ref[...], preferred_element_type=jnp.float32)
       + b_ref[...])
  o_ref[...] = jnp.maximum(z, 0.0)


def _l2_body(p_ref, h1_ref, wr_ref, wn_ref, b_ref, bt_ref,
             h2_ref, sum_ref, cnt_ref):
  i = pl.program_id(0)
  R = h1_ref.shape[0]
  D = h1_ref.shape[1]
  a = p_ref[0] + p_ref[1]
  h2 = (jnp.dot(h1_ref[...], wr_ref[...], preferred_element_type=jnp.float32)
        + jnp.dot(a, wn_ref[...], preferred_element_type=jnp.float32)
        + b_ref[...])
  h2_ref[...] = h2
  gids = lax.broadcasted_iota(jnp.int32, (R, G), 1)
  onehot = (bt_ref[...] == gids).astype(jnp.float32)
  ps = lax.dot_general(onehot, h2, (((0,), (0,)), ((), ())),
                       preferred_element_type=jnp.float32)
  pc = lax.dot_general(onehot, jnp.ones((R, D), jnp.float32),
                       (((0,), (0,)), ((), ())),
                       preferred_element_type=jnp.float32)

  @pl.when(i == 0)
  def _():
    sum_ref[...] = jnp.zeros_like(sum_ref)
    cnt_ref[...] = jnp.zeros_like(cnt_ref)

  sum_ref[...] += ps
  cnt_ref[...] += pc


def _rf_body(h2_ref, bt_ref, sum_ref, cnt_ref, o_ref):
  R = h2_ref.shape[0]
  mean = sum_ref[...] / jnp.maximum(cnt_ref[...], 1.0)
  gids = lax.broadcasted_iota(jnp.int32, (R, G), 1)
  onehot = (bt_ref[...] == gids).astype(jnp.float32)
  o_ref[...] = h2_ref[...] + jnp.dot(onehot, mean,
                                     preferred_element_type=jnp.float32)


def kernel(h, group_idx, batch, W1_root, W1_nbr, b1, W3_root, W3_nbr, b3):
  N, D = h.shape
  E = group_idx.shape[1]
  NW = NC * NS
  ch = -(-E // (NW * CB))
  e_pad = NW * ch * CB
  n_pad = -(-(N + 1) // (NS * 8)) * (NS * 8)
  rows_per_sub = n_pad // NS
  R = 2000
  grid = N // R

  src = group_idx[0]
  dst = group_idx[1]
  src_r = jnp.pad(src, (0, e_pad - E), constant_values=N).reshape(NW, ch, CB)
  dst_r = jnp.pad(dst, (0, e_pad - E), constant_values=N).reshape(NW, ch, CB)
  h_pad = jnp.pad(h, ((0, n_pad - N), (0, 0)))
  zeros = jnp.zeros((rows_per_sub, D), jnp.float32)
  bt = batch.reshape(N, 1)

  segsum = _make_segsum(n_pad, D, ch)

  p1 = segsum(h_pad, zeros, src_r, dst_r)
  h1_pad = pl.pallas_call(
      _l1_body,
      grid=(grid,),
      in_specs=[
          pl.BlockSpec((NC, R, D), lambda i: (0, i, 0)),
          pl.BlockSpec((R, D), lambda i: (i, 0)),
          pl.BlockSpec((D, D), lambda i: (0, 0)),
          pl.BlockSpec((D, D), lambda i: (0, 0)),
          pl.BlockSpec((1, D), lambda i: (0, 0)),
      ],
      out_specs=pl.BlockSpec((R, D), lambda i: (i, 0)),
      out_shape=jax.ShapeDtypeStruct((n_pad, D), jnp.float32),
  )(p1, h, W1_root, W1_nbr, b1.reshape(1, D))

  p2 = segsum(h1_pad, zeros, src_r, dst_r)
  h2, sums, counts = pl.pallas_call(
      _l2_body,
      grid=(grid,),
      in_specs=[
          pl.BlockSpec((NC, R, D), lambda i: (0, i, 0)),
          pl.BlockSpec((R, D), lambda i: (i, 0)),
          pl.BlockSpec((D, D), lambda i: (0, 0)),
          pl.BlockSpec((D, D), lambda i: (0, 0)),
          pl.BlockSpec((1, D), lambda i: (0, 0)),
          pl.BlockSpec((R, 1), lambda i: (i, 0)),
      ],
      out_specs=(
          pl.BlockSpec((R, D), lambda i: (i, 0)),
          pl.BlockSpec((G, D), lambda i: (0, 0)),
          pl.BlockSpec((G, D), lambda i: (0, 0)),
      ),
      out_shape=(
          jax.ShapeDtypeStruct((N, D), jnp.float32),
          jax.ShapeDtypeStruct((G, D), jnp.float32),
          jax.ShapeDtypeStruct((G, D), jnp.float32),
      ),
  )(p2, h1_pad, W3_root, W3_nbr, b3.reshape(1, D), bt)

  out = pl.pallas_call(
      _rf_body,
      grid=(grid,),
      in_specs=[
          pl.BlockSpec((R, D), lambda i: (i, 0)),
          pl.BlockSpec((R, 1), lambda i: (i, 0)),
          pl.BlockSpec((G, D), lambda i: (0, 0)),
          pl.BlockSpec((G, D), lambda i: (0, 0)),
      ],
      out_specs=pl.BlockSpec((R, D), lambda i: (i, 0)),
      out_shape=jax.ShapeDtypeStruct((N, D), jnp.float32),
  )(h2, bt, sums, counts)
  return out
